# Initial kernel scaffold; baseline (speedup 1.0000x reference)
import jax
import jax.numpy as jnp
from jax.experimental import pallas as pl


def kernel(x):
    raise NotImplementedError("write your pallas kernel here")


# speedup vs baseline: 3119.0316x; 3119.0316x over previous
import numpy as np
import jax
import jax.numpy as jnp
from jax.experimental import pallas as pl
from jax.experimental.pallas import tpu as pltpu

_ALPHA_K = np.array([0.296, 0.5567, 0.7088, 1.1286], dtype=np.float32)


def _codebook():
    n = len(_ALPHA_K)
    signs = np.array(np.meshgrid(*([[-1.0, 1.0]] * n), indexing="ij")).reshape(n, -1).T
    q = np.sort((signs * _ALPHA_K[None, :]).sum(axis=1)).astype(np.float32)
    edges = (0.5 * (q[1:] + q[:-1])).astype(np.float32)
    return q, edges


_Q, _EDGES = _codebook()
_E_POS = [float(v) for v in _EDGES[8:15]]
_GAPS = [float(v) for v in np.diff(_Q)[8:15]]
_Q_BASE = float(_Q[8])

_LANES = 128
_ROWS = 4704
_KSTEPS = 8


def _stats_body(x_ref, c_ref, acc_ref):
    k = pl.program_id(1)
    nk = pl.num_programs(1)

    @pl.when(k == 0)
    def _():
        acc_ref[0] = 0.0
        acc_ref[1] = 0.0

    blk = x_ref[0]
    acc_ref[0] += jnp.sum(blk)
    acc_ref[1] += jnp.sum(blk * blk)

    @pl.when(k == nk - 1)
    def _():
        n = jnp.float32(_ROWS * _KSTEPS * _LANES)
        s1 = acc_ref[0]
        s2 = acc_ref[1]
        mean = s1 / n
        var = (s2 - s1 * mean) / (n - 1.0)
        raw_std = jnp.sqrt(jnp.maximum(var, 0.0))
        std = raw_std + 1e-12
        deg = raw_std < 1e-12
        zero = jnp.float32(0.0)
        for i in range(7):
            c_ref[0, 0, i] = _E_POS[i] * std
        c_ref[0, 0, 7] = jnp.where(deg, zero, _Q_BASE * std)
        for i in range(7):
            c_ref[0, 0, 8 + i] = jnp.where(deg, zero, _GAPS[i] * std)
        c_ref[0, 0, 15] = jnp.where(deg, zero, mean)


def _apply_body(c_ref, x_ref, o_ref):
    s = pl.program_id(0)
    mu = c_ref[s, 0, 15]
    t = x_ref[0] - mu
    a = jnp.abs(t)
    f = jnp.where(a > c_ref[s, 0, 0], c_ref[s, 0, 8], 0.0)
    for i in range(1, 7):
        f = f + jnp.where(a > c_ref[s, 0, i], c_ref[s, 0, 8 + i], 0.0)
    f = f + c_ref[s, 0, 7]
    o_ref[0] = jnp.where(t < 0.0, -f, f) + mu


def kernel(x):
    b, c, h, w = x.shape
    n = c * h * w
    xr = x.reshape(b, n // _LANES, _LANES)

    coeffs = pl.pallas_call(
        _stats_body,
        grid=(b, _KSTEPS),
        in_specs=[pl.BlockSpec((1, _ROWS, _LANES), lambda s, k: (s, k, 0))],
        out_specs=pl.BlockSpec(
            (1, 1, 16), lambda s, k: (s, 0, 0), memory_space=pltpu.SMEM
        ),
        out_shape=jax.ShapeDtypeStruct((b, 1, 16), jnp.float32),
        scratch_shapes=[pltpu.SMEM((2,), jnp.float32)],
    )(xr)

    out = pl.pallas_call(
        _apply_body,
        grid=(b, _KSTEPS),
        in_specs=[
            pl.BlockSpec(memory_space=pltpu.SMEM),
            pl.BlockSpec((1, _ROWS, _LANES), lambda s, k: (s, k, 0)),
        ],
        out_specs=pl.BlockSpec((1, _ROWS, _LANES), lambda s, k: (s, k, 0)),
        out_shape=jax.ShapeDtypeStruct(xr.shape, jnp.float32),
    )(coeffs, xr)

    return out.reshape(b, c, h, w)
